# baseline (speedup 1.0000x reference)
import jax
import jax.numpy as jnp
from jax.experimental import pallas as pl


_BLOCK_ROWS = 1024


def _persite_kernel(x_ref, m_ref, w_ref, b_ref, t_ref, o_ref):
    x = x_ref[...]
    w = w_ref[...]
    r = jnp.dot(x, w, preferred_element_type=jnp.float32)
    o_ref[...] = (r + b_ref[0, 0]) * m_ref[...] * jnp.exp(t_ref[...])


def kernel(encoded_parents, masks, W, b, log_site_table):
    B, L, D = encoded_parents.shape
    S = log_site_table.shape[0]
    br = _BLOCK_ROWS
    rows = B * L
    x2 = encoded_parents.reshape(rows, D)
    m2 = masks.reshape(rows, 1)
    b2 = b.reshape(1, 1)
    nblk = rows // br
    blocks_per_seq = L // br

    out = pl.pallas_call(
        _persite_kernel,
        grid=(nblk,),
        in_specs=[
            pl.BlockSpec((br, D), lambda i: (i, 0)),
            pl.BlockSpec((br, 1), lambda i: (i, 0)),
            pl.BlockSpec((D, 1), lambda i: (0, 0)),
            pl.BlockSpec((1, 1), lambda i: (0, 0)),
            pl.BlockSpec((br, 1), lambda i: (i % blocks_per_seq, 0)),
        ],
        out_specs=pl.BlockSpec((br, 1), lambda i: (i, 0)),
        out_shape=jax.ShapeDtypeStruct((rows, 1), jnp.float32),
    )(x2, m2, W, b2, log_site_table)
    return out.reshape(B, L)


# speedup vs baseline: 1.0793x; 1.0793x over previous
import jax
import jax.numpy as jnp
from jax.experimental import pallas as pl
from jax.experimental.pallas import tpu as pltpu


_BLOCK_ROWS = 2048


def _persite_kernel(x_ref, m_ref, w_ref, b_ref, t_ref, o_ref):
    x = x_ref[...]
    w = w_ref[...]
    r = jnp.dot(x, w, preferred_element_type=jnp.float32)
    o_ref[...] = (r + b_ref[0, 0]) * m_ref[...] * jnp.exp(t_ref[...])


def kernel(encoded_parents, masks, W, b, log_site_table):
    B, L, D = encoded_parents.shape
    S = log_site_table.shape[0]
    br = _BLOCK_ROWS
    rows = B * L
    x2 = encoded_parents.reshape(rows, D)
    m2 = masks.reshape(rows, 1)
    b2 = b.reshape(1, 1)
    nblk = rows // br
    blocks_per_seq = L // br

    out = pl.pallas_call(
        _persite_kernel,
        grid=(nblk,),
        in_specs=[
            pl.BlockSpec((br, D), lambda i: (i, 0)),
            pl.BlockSpec((br, 1), lambda i: (i, 0)),
            pl.BlockSpec((D, 1), lambda i: (0, 0)),
            pl.BlockSpec((1, 1), lambda i: (0, 0)),
            pl.BlockSpec((br, 1), lambda i: (i % blocks_per_seq, 0)),
        ],
        out_specs=pl.BlockSpec((br, 1), lambda i: (i, 0)),
        out_shape=jax.ShapeDtypeStruct((rows, 1), jnp.float32),
        compiler_params=pltpu.CompilerParams(
            dimension_semantics=("parallel",),
        ),
    )(x2, m2, W, b2, log_site_table)
    return out.reshape(B, L)


# speedup vs baseline: 1.0818x; 1.0023x over previous
import jax
import jax.numpy as jnp
from jax.experimental import pallas as pl
from jax.experimental.pallas import tpu as pltpu


_BLOCK_ROWS = 2048


def _persite_kernel(x_ref, m_ref, w_ref, b_ref, t_ref, o_ref):
    x = x_ref[0]
    w = w_ref[...]
    r = jnp.dot(x, w, preferred_element_type=jnp.float32)
    o_ref[0] = (r + b_ref[0, 0]) * m_ref[0] * jnp.exp(t_ref[...])


def kernel(encoded_parents, masks, W, b, log_site_table):
    B, L, D = encoded_parents.shape
    bl = _BLOCK_ROWS
    m3 = masks.reshape(B, L, 1)
    b2 = b.reshape(1, 1)

    out = pl.pallas_call(
        _persite_kernel,
        grid=(B, L // bl),
        in_specs=[
            pl.BlockSpec((1, bl, D), lambda i, j: (i, j, 0)),
            pl.BlockSpec((1, bl, 1), lambda i, j: (i, j, 0)),
            pl.BlockSpec((D, 1), lambda i, j: (0, 0)),
            pl.BlockSpec((1, 1), lambda i, j: (0, 0)),
            pl.BlockSpec((bl, 1), lambda i, j: (j, 0)),
        ],
        out_specs=pl.BlockSpec((1, bl, 1), lambda i, j: (i, j, 0)),
        out_shape=jax.ShapeDtypeStruct((B, L, 1), jnp.float32),
        compiler_params=pltpu.CompilerParams(
            dimension_semantics=("parallel", "parallel"),
        ),
    )(encoded_parents, m3, W, b2, log_site_table)
    return out.reshape(B, L)
